# baseline (speedup 1.0000x reference)
import jax
import jax.numpy as jnp
from jax.experimental import pallas as pl
from jax.experimental.pallas import tpu as pltpu

SEQ = 4096
FEAT = 26
DIM = 64
BATCH = 4

BS = 256


def _add_body(x_ref, t_ref, o_ref):
    t3 = t_ref[...].reshape(1, BS, FEAT, DIM)
    o_ref[...] = x_ref[...] + t3


def kernel(inputs, table):
    return pl.pallas_call(
        _add_body,
        grid=(SEQ // BS, BATCH),
        in_specs=[
            pl.BlockSpec((1, BS, FEAT, DIM), lambda s, b: (b, s, 0, 0)),
            pl.BlockSpec((BS * FEAT, DIM), lambda s, b: (s, 0)),
        ],
        out_specs=pl.BlockSpec((1, BS, FEAT, DIM), lambda s, b: (b, s, 0, 0)),
        out_shape=jax.ShapeDtypeStruct((BATCH, SEQ, FEAT, DIM), jnp.float32),
        compiler_params=pltpu.CompilerParams(
            dimension_semantics=("arbitrary", "arbitrary"),
        ),
    )(inputs, table)
